# baseline (speedup 1.0000x reference)
import functools

import jax
import jax.numpy as jnp
from jax import lax
from jax.experimental import pallas as pl
from jax.experimental.pallas import tpu as pltpu
from jax.experimental.pallas import tpu_sc as plsc

B, N = 4, 256
T = B * N
IN_DIM, D, QD = 4096, 1024, 1024
E, K, DFF = 8, 2, 2048
CAP = T
SLOTS = E * CAP
NBLK = CAP // 128
NTILES = 32
TPW = T // NTILES


def _bdot(a, b):
    return jnp.dot(a.astype(jnp.bfloat16), b.astype(jnp.bfloat16),
                   preferred_element_type=jnp.float32)


def _gelu(x):
    return 0.5 * x * (1.0 + jax.lax.erf(x * 0.7071067811865476))


def _mlp1_body(vis_ref, q_ref, lng_ref, lnb_ref, win_ref, bin_ref,
               wout_ref, bout_ref, wrx_ref, wrq_ref, br_ref,
               xbf_ref, s1_ref, s2_ref, g1_ref, g2_ref,
               mask_sum_ref, p_sum_ref, z_sum_ref, nb_ref):
    i = pl.program_id(0)
    v = vis_ref[...]
    mu = jnp.mean(v, axis=-1, keepdims=True)
    var = jnp.mean((v - mu) ** 2, axis=-1, keepdims=True)
    ln = (v - mu) / jnp.sqrt(var + 1e-5) * lng_ref[...] + lnb_ref[...]
    h = _gelu(_bdot(ln, win_ref[...]) + bin_ref[...])
    x = _bdot(h, wout_ref[...]) + bout_ref[...]
    xbf_ref[...] = x.astype(jnp.bfloat16)
    qlog = _bdot(q_ref[...].reshape(1, QD), wrq_ref[...])
    logits = _bdot(x, wrx_ref[...]) + qlog + br_ref[...]

    m = jnp.max(logits, axis=-1, keepdims=True)
    unnorm = jnp.exp(logits - m)
    denom = jnp.sum(unnorm, axis=-1, keepdims=True)
    probs = unnorm / denom
    lse = jnp.log(denom) + m

    eio = jax.lax.broadcasted_iota(jnp.int32, (N, E), 1)
    m1 = jnp.max(probs, axis=-1, keepdims=True)
    i1 = jnp.min(jnp.where(probs == m1, eio, E), axis=-1, keepdims=True)
    masked = jnp.where(eio == i1, -1.0, probs)
    m2 = jnp.max(masked, axis=-1, keepdims=True)
    i2 = jnp.min(jnp.where(masked == m2, eio, E), axis=-1, keepdims=True)
    tot = m1 + m2
    g1_ref[...] = m1 / tot
    g2_ref[...] = m2 / tot
    sel1 = eio == i1
    sel2 = eio == i2
    mask = sel1.astype(jnp.float32) + sel2.astype(jnp.float32)

    @pl.when(i == 0)
    def _():
        mask_sum_ref[...] = jnp.zeros_like(mask_sum_ref)
        p_sum_ref[...] = jnp.zeros_like(p_sum_ref)
        z_sum_ref[...] = jnp.zeros_like(z_sum_ref)

    rio = jax.lax.broadcasted_iota(jnp.int32, (N, N), 0)
    cio = jax.lax.broadcasted_iota(jnp.int32, (N, N), 1)
    lt = (cio < rio).astype(jnp.bfloat16)
    cum = jnp.dot(lt, mask.astype(jnp.bfloat16),
                  preferred_element_type=jnp.float32)
    rmat = mask_sum_ref[...] + cum
    r1 = jnp.sum(jnp.where(sel1, rmat, 0.0), axis=-1, keepdims=True)
    r2 = jnp.sum(jnp.where(sel2, rmat, 0.0), axis=-1, keepdims=True)
    s1_ref[...] = i1 * CAP + r1.astype(jnp.int32)
    s2_ref[...] = i2 * CAP + r2.astype(jnp.int32)

    mask_sum_ref[...] += jnp.sum(mask, axis=0, keepdims=True)
    p_sum_ref[...] += jnp.sum(probs, axis=0, keepdims=True)
    z_sum_ref[...] += jnp.sum(lse * lse).reshape(1, 1)
    cnt = mask_sum_ref[...].astype(jnp.int32)
    nb_ref[...] = jax.lax.shift_right_logical(cnt + 127, 7)


def _mlp1_router(vis, query, ln_g, ln_b, w_in, b_in, w_out, b_out,
                 w_rx, w_rq, b_r):
    return pl.pallas_call(
        _mlp1_body,
        grid=(B,),
        in_specs=[
            pl.BlockSpec((N, IN_DIM), lambda i: (i, 0)),
            pl.BlockSpec((1, 1, QD), lambda i: (i, 0, 0)),
            pl.BlockSpec((1, IN_DIM), lambda i: (0, 0)),
            pl.BlockSpec((1, IN_DIM), lambda i: (0, 0)),
            pl.BlockSpec((IN_DIM, D), lambda i: (0, 0)),
            pl.BlockSpec((1, D), lambda i: (0, 0)),
            pl.BlockSpec((D, D), lambda i: (0, 0)),
            pl.BlockSpec((1, D), lambda i: (0, 0)),
            pl.BlockSpec((D, E), lambda i: (0, 0)),
            pl.BlockSpec((QD, E), lambda i: (0, 0)),
            pl.BlockSpec((1, E), lambda i: (0, 0)),
        ],
        out_specs=[
            pl.BlockSpec((N, D), lambda i: (i, 0)),
            pl.BlockSpec((N, 1), lambda i: (i, 0)),
            pl.BlockSpec((N, 1), lambda i: (i, 0)),
            pl.BlockSpec((N, 1), lambda i: (i, 0)),
            pl.BlockSpec((N, 1), lambda i: (i, 0)),
            pl.BlockSpec((1, E), lambda i: (0, 0)),
            pl.BlockSpec((1, E), lambda i: (0, 0)),
            pl.BlockSpec((1, 1), lambda i: (0, 0)),
            pl.BlockSpec((1, E), lambda i: (0, 0)),
        ],
        out_shape=[
            jax.ShapeDtypeStruct((T, D), jnp.bfloat16),
            jax.ShapeDtypeStruct((T, 1), jnp.int32),
            jax.ShapeDtypeStruct((T, 1), jnp.int32),
            jax.ShapeDtypeStruct((T, 1), jnp.float32),
            jax.ShapeDtypeStruct((T, 1), jnp.float32),
            jax.ShapeDtypeStruct((1, E), jnp.float32),
            jax.ShapeDtypeStruct((1, E), jnp.float32),
            jax.ShapeDtypeStruct((1, 1), jnp.float32),
            jax.ShapeDtypeStruct((1, E), jnp.int32),
        ],
    )(vis, query.reshape(B, 1, QD), ln_g.reshape(1, IN_DIM),
      ln_b.reshape(1, IN_DIM), w_in, b_in.reshape(1, D), w_out,
      b_out.reshape(1, D), w_rx, w_rq, b_r.reshape(1, E))


_SC_MESH = plsc.VectorSubcoreMesh(core_axis_name="c", subcore_axis_name="s",
                                  num_cores=2, num_subcores=16)


def _scatter_body(x3_hbm, s1_hbm, s2_hbm, xg_hbm, idx1_v, idx2_v, rows_v,
                  sem):
    wid = lax.axis_index("s") * 2 + lax.axis_index("c")
    base = wid * TPW
    pltpu.sync_copy(s1_hbm.at[pl.ds(base, TPW)], idx1_v)
    pltpu.sync_copy(s2_hbm.at[pl.ds(base, TPW)], idx2_v)
    pltpu.sync_copy(x3_hbm.at[pl.ds(base, TPW)], rows_v)
    pltpu.async_copy(rows_v, xg_hbm.at[idx1_v], sem).wait()
    pltpu.async_copy(rows_v, xg_hbm.at[idx2_v], sem).wait()


def _sc_scatter(x3, s1, s2):
    return pl.kernel(
        _scatter_body,
        out_type=jax.ShapeDtypeStruct((SLOTS, 4, 128), jnp.int32),
        mesh=_SC_MESH,
        scratch_types=[
            pltpu.VMEM((TPW,), jnp.int32),
            pltpu.VMEM((TPW,), jnp.int32),
            pltpu.VMEM((TPW, 4, 128), jnp.int32),
            pltpu.SemaphoreType.DMA,
        ],
    )(x3, s1, s2)


def _ffn_body(nb_ref, xg_ref, w1_ref, b1_ref, w2_ref, b2_ref, yg_ref):
    e = pl.program_id(0)
    j = pl.program_id(1)

    @pl.when(j < nb_ref[e])
    def _():
        x = xg_ref[...]
        h = _gelu(jnp.dot(x, w1_ref[...].reshape(D, DFF).astype(jnp.bfloat16),
                          preferred_element_type=jnp.float32)
                  + b1_ref[...].reshape(1, DFF))
        y = jnp.dot(h.astype(jnp.bfloat16),
                    w2_ref[...].reshape(DFF, D).astype(jnp.bfloat16),
                    preferred_element_type=jnp.float32)
        yg_ref[...] = y + b2_ref[...].reshape(1, D)


def _ffn(nb, xg, W1, b1, W2, b2):
    def xg_map(e, j, nb_ref):
        j_eff = jnp.minimum(j, jnp.maximum(nb_ref[e] - 1, 0))
        return (e * NBLK + j_eff, 0)

    grid_spec = pltpu.PrefetchScalarGridSpec(
        num_scalar_prefetch=1,
        grid=(E, NBLK),
        in_specs=[
            pl.BlockSpec((128, D), xg_map),
            pl.BlockSpec((1, D, DFF), lambda e, j, nb_ref: (e, 0, 0)),
            pl.BlockSpec((1, 1, DFF), lambda e, j, nb_ref: (e, 0, 0)),
            pl.BlockSpec((1, DFF, D), lambda e, j, nb_ref: (e, 0, 0)),
            pl.BlockSpec((1, 1, D), lambda e, j, nb_ref: (e, 0, 0)),
        ],
        out_specs=pl.BlockSpec((128, D), lambda e, j, nb_ref: (e * NBLK + j, 0)),
    )
    return pl.pallas_call(
        _ffn_body,
        grid_spec=grid_spec,
        out_shape=jax.ShapeDtypeStruct((SLOTS, D), jnp.float32),
    )(nb, xg, W1, b1.reshape(E, 1, DFF), W2, b2.reshape(E, 1, D))


def _gather_body(yg_hbm, s1_hbm, s2_hbm, y1_hbm, y2_hbm, idx_v, buf_v, sem):
    wid = lax.axis_index("s") * 2 + lax.axis_index("c")
    base = wid * TPW
    pltpu.sync_copy(s1_hbm.at[pl.ds(base, TPW)], idx_v)
    pltpu.async_copy(yg_hbm.at[idx_v], buf_v, sem).wait()
    pltpu.sync_copy(buf_v, y1_hbm.at[pl.ds(base, TPW)])
    pltpu.sync_copy(s2_hbm.at[pl.ds(base, TPW)], idx_v)
    pltpu.async_copy(yg_hbm.at[idx_v], buf_v, sem).wait()
    pltpu.sync_copy(buf_v, y2_hbm.at[pl.ds(base, TPW)])


def _sc_gather(yg, s1, s2):
    return pl.kernel(
        _gather_body,
        out_type=[
            jax.ShapeDtypeStruct((T, D), jnp.float32),
            jax.ShapeDtypeStruct((T, D), jnp.float32),
        ],
        mesh=_SC_MESH,
        scratch_types=[
            pltpu.VMEM((TPW,), jnp.int32),
            pltpu.VMEM((TPW, D), jnp.float32),
            pltpu.SemaphoreType.DMA,
        ],
    )(yg, s1, s2)


def _combine_body(y1_ref, y2_ref, g1_ref, g2_ref, msum_ref, psum_ref,
                  zsum_ref, out_ref, lb_ref, z_ref):
    out_ref[...] = g1_ref[...] * y1_ref[...] + g2_ref[...] * y2_ref[...]
    f = msum_ref[...] * (1.0 / T)
    pm = psum_ref[...] * (1.0 / T)
    lb_ref[...] = (E * jnp.sum(f * pm)).reshape(1, 1)
    z_ref[...] = zsum_ref[...] * (1.0 / T)


def _combine(y1, y2, g1, g2, msum, psum, zsum):
    return pl.pallas_call(
        _combine_body,
        out_shape=[
            jax.ShapeDtypeStruct((T, D), jnp.float32),
            jax.ShapeDtypeStruct((1, 1), jnp.float32),
            jax.ShapeDtypeStruct((1, 1), jnp.float32),
        ],
    )(y1, y2, g1, g2, msum, psum, zsum)


def kernel(vis_emb, query_emb, ln_g, ln_b, w_in, b_in, w_out, b_out,
           w_r, b_r, W1, b1, W2, b2):
    vis = vis_emb.reshape(T, IN_DIM)
    (xbf, s1, s2, g1, g2, msum, psum, zsum, nb) = _mlp1_router(
        vis, query_emb, ln_g, ln_b, w_in, b_in, w_out, b_out,
        w_r[:D], w_r[D:], b_r)
    s1f = s1.reshape(T)
    s2f = s2.reshape(T)
    x3i = jax.lax.bitcast_convert_type(
        xbf.reshape(T, 512, 2), jnp.int32).reshape(T, 4, 128)
    xgi = _sc_scatter(x3i, s1f, s2f)
    xg = jax.lax.bitcast_convert_type(
        xgi, jnp.bfloat16).reshape(SLOTS, D)
    yg = _ffn(nb.reshape(E), xg, W1, b1, W2, b2)
    y1, y2 = _sc_gather(yg, s1f, s2f)
    out, lb, z = _combine(y1, y2, g1, g2, msum, psum, zsum)
    return (out.reshape(B, N, D), lb.reshape(()), z.reshape(()))


# speedup vs baseline: 1.8283x; 1.8283x over previous
import functools

import jax
import jax.numpy as jnp
from jax import lax
from jax.experimental import pallas as pl
from jax.experimental.pallas import tpu as pltpu
from jax.experimental.pallas import tpu_sc as plsc

B, N = 4, 256
T = B * N
IN_DIM, D, QD = 4096, 1024, 1024
E, K, DFF = 8, 2, 2048
CAP = T
SLOTS = E * CAP
NBLK = CAP // 128
NTILES = 32
TPW = T // NTILES


def _bdot(a, b):
    return jnp.dot(a.astype(jnp.bfloat16), b.astype(jnp.bfloat16),
                   preferred_element_type=jnp.float32)


def _gelu(x):
    return 0.5 * x * (1.0 + jax.lax.erf(x * 0.7071067811865476))


def _mlp1_body(vis_ref, q_ref, lng_ref, lnb_ref, win_ref, bin_ref,
               wout_ref, bout_ref, wrx_ref, wrq_ref, br_ref,
               xbf_ref, s1_ref, s2_ref, g1_ref, g2_ref,
               mask_sum_ref, p_sum_ref, z_sum_ref, nb_ref):
    i = pl.program_id(0)
    v = vis_ref[...]
    mu = jnp.mean(v, axis=-1, keepdims=True)
    var = jnp.mean((v - mu) ** 2, axis=-1, keepdims=True)
    ln = (v - mu) / jnp.sqrt(var + 1e-5) * lng_ref[...] + lnb_ref[...]
    h = _gelu(_bdot(ln, win_ref[...]) + bin_ref[...])
    x = _bdot(h, wout_ref[...]) + bout_ref[...]
    xbf_ref[...] = x
    qlog = _bdot(q_ref[...].reshape(1, QD), wrq_ref[...])
    logits = _bdot(x, wrx_ref[...]) + qlog + br_ref[...]

    m = jnp.max(logits, axis=-1, keepdims=True)
    unnorm = jnp.exp(logits - m)
    denom = jnp.sum(unnorm, axis=-1, keepdims=True)
    probs = unnorm / denom
    lse = jnp.log(denom) + m

    eio = jax.lax.broadcasted_iota(jnp.int32, (N, E), 1)
    m1 = jnp.max(probs, axis=-1, keepdims=True)
    i1 = jnp.min(jnp.where(probs == m1, eio, E), axis=-1, keepdims=True)
    masked = jnp.where(eio == i1, -1.0, probs)
    m2 = jnp.max(masked, axis=-1, keepdims=True)
    i2 = jnp.min(jnp.where(masked == m2, eio, E), axis=-1, keepdims=True)
    tot = m1 + m2
    g1_ref[...] = m1 / tot
    g2_ref[...] = m2 / tot
    sel1 = eio == i1
    sel2 = eio == i2
    mask = sel1.astype(jnp.float32) + sel2.astype(jnp.float32)

    @pl.when(i == 0)
    def _():
        mask_sum_ref[...] = jnp.zeros_like(mask_sum_ref)
        p_sum_ref[...] = jnp.zeros_like(p_sum_ref)
        z_sum_ref[...] = jnp.zeros_like(z_sum_ref)

    rio = jax.lax.broadcasted_iota(jnp.int32, (N, N), 0)
    cio = jax.lax.broadcasted_iota(jnp.int32, (N, N), 1)
    lt = (cio < rio).astype(jnp.bfloat16)
    cum = jnp.dot(lt, mask.astype(jnp.bfloat16),
                  preferred_element_type=jnp.float32)
    rmat = mask_sum_ref[...] + cum
    r1 = jnp.sum(jnp.where(sel1, rmat, 0.0), axis=-1, keepdims=True)
    r2 = jnp.sum(jnp.where(sel2, rmat, 0.0), axis=-1, keepdims=True)
    s1_ref[...] = i1 * CAP + r1.astype(jnp.int32)
    s2_ref[...] = i2 * CAP + r2.astype(jnp.int32)

    mask_sum_ref[...] += jnp.sum(mask, axis=0, keepdims=True)
    p_sum_ref[...] += jnp.sum(probs, axis=0, keepdims=True)
    z_sum_ref[...] += jnp.sum(lse * lse).reshape(1, 1)
    cnt = mask_sum_ref[...].astype(jnp.int32)
    nb_ref[...] = jax.lax.shift_right_logical(cnt + 127, 7)


def _mlp1_router(vis, query, ln_g, ln_b, w_in, b_in, w_out, b_out,
                 w_rx, w_rq, b_r):
    return pl.pallas_call(
        _mlp1_body,
        grid=(B,),
        in_specs=[
            pl.BlockSpec((N, IN_DIM), lambda i: (i, 0)),
            pl.BlockSpec((1, 1, QD), lambda i: (i, 0, 0)),
            pl.BlockSpec((1, IN_DIM), lambda i: (0, 0)),
            pl.BlockSpec((1, IN_DIM), lambda i: (0, 0)),
            pl.BlockSpec((IN_DIM, D), lambda i: (0, 0)),
            pl.BlockSpec((1, D), lambda i: (0, 0)),
            pl.BlockSpec((D, D), lambda i: (0, 0)),
            pl.BlockSpec((1, D), lambda i: (0, 0)),
            pl.BlockSpec((D, E), lambda i: (0, 0)),
            pl.BlockSpec((QD, E), lambda i: (0, 0)),
            pl.BlockSpec((1, E), lambda i: (0, 0)),
        ],
        out_specs=[
            pl.BlockSpec((N, D), lambda i: (i, 0)),
            pl.BlockSpec((N, 1), lambda i: (i, 0)),
            pl.BlockSpec((N, 1), lambda i: (i, 0)),
            pl.BlockSpec((N, 1), lambda i: (i, 0)),
            pl.BlockSpec((N, 1), lambda i: (i, 0)),
            pl.BlockSpec((1, E), lambda i: (0, 0)),
            pl.BlockSpec((1, E), lambda i: (0, 0)),
            pl.BlockSpec((1, 1), lambda i: (0, 0)),
            pl.BlockSpec((1, E), lambda i: (0, 0)),
        ],
        out_shape=[
            jax.ShapeDtypeStruct((T, D), jnp.float32),
            jax.ShapeDtypeStruct((T, 1), jnp.int32),
            jax.ShapeDtypeStruct((T, 1), jnp.int32),
            jax.ShapeDtypeStruct((T, 1), jnp.float32),
            jax.ShapeDtypeStruct((T, 1), jnp.float32),
            jax.ShapeDtypeStruct((1, E), jnp.float32),
            jax.ShapeDtypeStruct((1, E), jnp.float32),
            jax.ShapeDtypeStruct((1, 1), jnp.float32),
            jax.ShapeDtypeStruct((1, E), jnp.int32),
        ],
    )(vis, query.reshape(B, 1, QD), ln_g.reshape(1, IN_DIM),
      ln_b.reshape(1, IN_DIM), w_in, b_in.reshape(1, D), w_out,
      b_out.reshape(1, D), w_rx, w_rq, b_r.reshape(1, E))


_SC_MESH = plsc.VectorSubcoreMesh(core_axis_name="c", subcore_axis_name="s",
                                  num_cores=2, num_subcores=16)


def _scatter_body(x3_hbm, s1_hbm, s2_hbm, xg_hbm, idx1_v, idx2_v, rows_v,
                  sem):
    wid = lax.axis_index("s") * 2 + lax.axis_index("c")
    base = wid * TPW
    pltpu.sync_copy(s1_hbm.at[pl.ds(base, TPW)], idx1_v)
    pltpu.sync_copy(s2_hbm.at[pl.ds(base, TPW)], idx2_v)
    pltpu.sync_copy(x3_hbm.at[pl.ds(base, TPW)], rows_v)
    pltpu.async_copy(rows_v, xg_hbm.at[idx1_v], sem).wait()
    pltpu.async_copy(rows_v, xg_hbm.at[idx2_v], sem).wait()


def _sc_scatter(x2, s1, s2):
    return pl.kernel(
        _scatter_body,
        out_type=jax.ShapeDtypeStruct((SLOTS, D), jnp.float32),
        mesh=_SC_MESH,
        scratch_types=[
            pltpu.VMEM((TPW,), jnp.int32),
            pltpu.VMEM((TPW,), jnp.int32),
            pltpu.VMEM((TPW, D), jnp.float32),
            pltpu.SemaphoreType.DMA,
        ],
    )(x2, s1, s2)


def _ffn_body(nb_ref, xg_ref, w1_ref, b1_ref, w2_ref, b2_ref, yg_ref):
    e = pl.program_id(0)
    j = pl.program_id(1)

    @pl.when(j < nb_ref[e])
    def _():
        x = xg_ref[...].astype(jnp.bfloat16)
        h = _gelu(jnp.dot(x, w1_ref[...].reshape(D, DFF).astype(jnp.bfloat16),
                          preferred_element_type=jnp.float32)
                  + b1_ref[...].reshape(1, DFF))
        y = jnp.dot(h.astype(jnp.bfloat16),
                    w2_ref[...].reshape(DFF, D).astype(jnp.bfloat16),
                    preferred_element_type=jnp.float32)
        yg_ref[...] = y + b2_ref[...].reshape(1, D)


def _ffn(nb, xg, W1, b1, W2, b2):
    def xg_map(e, j, nb_ref):
        j_eff = jnp.minimum(j, jnp.maximum(nb_ref[e] - 1, 0))
        return (e * NBLK + j_eff, 0)

    grid_spec = pltpu.PrefetchScalarGridSpec(
        num_scalar_prefetch=1,
        grid=(E, NBLK),
        in_specs=[
            pl.BlockSpec((128, D), xg_map),
            pl.BlockSpec((1, D, DFF), lambda e, j, nb_ref: (e, 0, 0)),
            pl.BlockSpec((1, 1, DFF), lambda e, j, nb_ref: (e, 0, 0)),
            pl.BlockSpec((1, DFF, D), lambda e, j, nb_ref: (e, 0, 0)),
            pl.BlockSpec((1, 1, D), lambda e, j, nb_ref: (e, 0, 0)),
        ],
        out_specs=pl.BlockSpec((128, D), lambda e, j, nb_ref: (e * NBLK + j, 0)),
    )
    return pl.pallas_call(
        _ffn_body,
        grid_spec=grid_spec,
        out_shape=jax.ShapeDtypeStruct((SLOTS, D), jnp.float32),
    )(nb, xg, W1, b1.reshape(E, 1, DFF), W2, b2.reshape(E, 1, D))


def _gather_body(yg_hbm, s1_hbm, s2_hbm, y1_hbm, y2_hbm, idx_v, buf_v, sem):
    wid = lax.axis_index("s") * 2 + lax.axis_index("c")
    base = wid * TPW
    pltpu.sync_copy(s1_hbm.at[pl.ds(base, TPW)], idx_v)
    pltpu.async_copy(yg_hbm.at[idx_v], buf_v, sem).wait()
    pltpu.sync_copy(buf_v, y1_hbm.at[pl.ds(base, TPW)])
    pltpu.sync_copy(s2_hbm.at[pl.ds(base, TPW)], idx_v)
    pltpu.async_copy(yg_hbm.at[idx_v], buf_v, sem).wait()
    pltpu.sync_copy(buf_v, y2_hbm.at[pl.ds(base, TPW)])


def _sc_gather(yg, s1, s2):
    return pl.kernel(
        _gather_body,
        out_type=[
            jax.ShapeDtypeStruct((T, D), jnp.float32),
            jax.ShapeDtypeStruct((T, D), jnp.float32),
        ],
        mesh=_SC_MESH,
        scratch_types=[
            pltpu.VMEM((TPW,), jnp.int32),
            pltpu.VMEM((TPW, D), jnp.float32),
            pltpu.SemaphoreType.DMA,
        ],
    )(yg, s1, s2)


def _combine_body(y1_ref, y2_ref, g1_ref, g2_ref, msum_ref, psum_ref,
                  zsum_ref, out_ref, lb_ref, z_ref):
    out_ref[...] = g1_ref[...] * y1_ref[...] + g2_ref[...] * y2_ref[...]
    f = msum_ref[...] * (1.0 / T)
    pm = psum_ref[...] * (1.0 / T)
    lb_ref[...] = (E * jnp.sum(f * pm)).reshape(1, 1)
    z_ref[...] = zsum_ref[...] * (1.0 / T)


def _combine(y1, y2, g1, g2, msum, psum, zsum):
    return pl.pallas_call(
        _combine_body,
        out_shape=[
            jax.ShapeDtypeStruct((T, D), jnp.float32),
            jax.ShapeDtypeStruct((1, 1), jnp.float32),
            jax.ShapeDtypeStruct((1, 1), jnp.float32),
        ],
    )(y1, y2, g1, g2, msum, psum, zsum)


def kernel(vis_emb, query_emb, ln_g, ln_b, w_in, b_in, w_out, b_out,
           w_r, b_r, W1, b1, W2, b2):
    vis = vis_emb.reshape(T, IN_DIM)
    (xbf, s1, s2, g1, g2, msum, psum, zsum, nb) = _mlp1_router(
        vis, query_emb, ln_g, ln_b, w_in, b_in, w_out, b_out,
        w_r[:D], w_r[D:], b_r)
    s1f = s1.reshape(T)
    s2f = s2.reshape(T)
    xg = _sc_scatter(xbf, s1f, s2f)
    yg = _ffn(nb.reshape(E), xg, W1, b1, W2, b2)
    y1, y2 = _sc_gather(yg, s1f, s2f)
    out, lb, z = _combine(y1, y2, g1, g2, msum, psum, zsum)
    return (out.reshape(B, N, D), lb.reshape(()), z.reshape(()))


# speedup vs baseline: 1.9905x; 1.0887x over previous
import functools

import jax
import jax.numpy as jnp
from jax import lax
from jax.experimental import pallas as pl
from jax.experimental.pallas import tpu as pltpu
from jax.experimental.pallas import tpu_sc as plsc

B, N = 4, 256
T = B * N
IN_DIM, D, QD = 4096, 1024, 1024
E, K, DFF = 8, 2, 2048
CAP = T
SLOTS = E * CAP
NBLK = CAP // 128
NTILES = 32
TPW = T // NTILES


def _bdot(a, b):
    return jnp.dot(a.astype(jnp.bfloat16), b.astype(jnp.bfloat16),
                   preferred_element_type=jnp.float32)


def _gelu(x):
    return 0.5 * x * (1.0 + jax.lax.erf(x * 0.7071067811865476))


def _mlp1_body(vis_ref, q_ref, lng_ref, lnb_ref, win_ref, bin_ref,
               wout_ref, bout_ref, wrx_ref, wrq_ref, br_ref,
               xbf_ref, s1_ref, s2_ref, g1_ref, g2_ref,
               mask_sum_ref, p_sum_ref, z_sum_ref, nb_ref):
    i = pl.program_id(0)
    v = vis_ref[...]
    mu = jnp.mean(v, axis=-1, keepdims=True)
    var = jnp.mean((v - mu) ** 2, axis=-1, keepdims=True)
    ln = (v - mu) / jnp.sqrt(var + 1e-5) * lng_ref[...] + lnb_ref[...]
    h = _gelu(_bdot(ln, win_ref[...]) + bin_ref[...])
    x = _bdot(h, wout_ref[...]) + bout_ref[...]
    xbf_ref[...] = x
    qlog = _bdot(q_ref[...].reshape(1, QD), wrq_ref[...])
    logits = _bdot(x, wrx_ref[...]) + qlog + br_ref[...]

    m = jnp.max(logits, axis=-1, keepdims=True)
    unnorm = jnp.exp(logits - m)
    denom = jnp.sum(unnorm, axis=-1, keepdims=True)
    probs = unnorm / denom
    lse = jnp.log(denom) + m

    eio = jax.lax.broadcasted_iota(jnp.int32, (N, E), 1)
    m1 = jnp.max(probs, axis=-1, keepdims=True)
    i1 = jnp.min(jnp.where(probs == m1, eio, E), axis=-1, keepdims=True)
    masked = jnp.where(eio == i1, -1.0, probs)
    m2 = jnp.max(masked, axis=-1, keepdims=True)
    i2 = jnp.min(jnp.where(masked == m2, eio, E), axis=-1, keepdims=True)
    tot = m1 + m2
    g1_ref[...] = m1 / tot
    g2_ref[...] = m2 / tot
    sel1 = eio == i1
    sel2 = eio == i2
    mask = sel1.astype(jnp.float32) + sel2.astype(jnp.float32)

    @pl.when(i == 0)
    def _():
        mask_sum_ref[...] = jnp.zeros_like(mask_sum_ref)
        p_sum_ref[...] = jnp.zeros_like(p_sum_ref)
        z_sum_ref[...] = jnp.zeros_like(z_sum_ref)

    rio = jax.lax.broadcasted_iota(jnp.int32, (N, N), 0)
    cio = jax.lax.broadcasted_iota(jnp.int32, (N, N), 1)
    lt = (cio < rio).astype(jnp.bfloat16)
    cum = jnp.dot(lt, mask.astype(jnp.bfloat16),
                  preferred_element_type=jnp.float32)
    rmat = mask_sum_ref[...] + cum
    r1 = jnp.sum(jnp.where(sel1, rmat, 0.0), axis=-1, keepdims=True)
    r2 = jnp.sum(jnp.where(sel2, rmat, 0.0), axis=-1, keepdims=True)
    s1_ref[...] = (i1 * CAP + r1.astype(jnp.int32)).reshape(N)
    s2_ref[...] = (i2 * CAP + r2.astype(jnp.int32)).reshape(N)

    mask_sum_ref[...] += jnp.sum(mask, axis=0, keepdims=True)
    p_sum_ref[...] += jnp.sum(probs, axis=0, keepdims=True)
    z_sum_ref[...] += jnp.sum(lse * lse).reshape(1, 1)
    cnt = mask_sum_ref[...].astype(jnp.int32)
    nb_ref[...] = jax.lax.shift_right_logical(cnt + 127, 7).reshape(E)


def _mlp1_router(vis, query, ln_g, ln_b, w_in, b_in, w_out, b_out,
                 w_rx, w_rq, b_r):
    return pl.pallas_call(
        _mlp1_body,
        grid=(B,),
        in_specs=[
            pl.BlockSpec((N, IN_DIM), lambda i: (i, 0)),
            pl.BlockSpec((1, 1, QD), lambda i: (i, 0, 0)),
            pl.BlockSpec((1, IN_DIM), lambda i: (0, 0)),
            pl.BlockSpec((1, IN_DIM), lambda i: (0, 0)),
            pl.BlockSpec((IN_DIM, D), lambda i: (0, 0)),
            pl.BlockSpec((1, D), lambda i: (0, 0)),
            pl.BlockSpec((D, D), lambda i: (0, 0)),
            pl.BlockSpec((1, D), lambda i: (0, 0)),
            pl.BlockSpec((D, E), lambda i: (0, 0)),
            pl.BlockSpec((QD, E), lambda i: (0, 0)),
            pl.BlockSpec((1, E), lambda i: (0, 0)),
        ],
        out_specs=[
            pl.BlockSpec((N, D), lambda i: (i, 0)),
            pl.BlockSpec((N,), lambda i: (i,)),
            pl.BlockSpec((N,), lambda i: (i,)),
            pl.BlockSpec((N, 1), lambda i: (i, 0)),
            pl.BlockSpec((N, 1), lambda i: (i, 0)),
            pl.BlockSpec((1, E), lambda i: (0, 0)),
            pl.BlockSpec((1, E), lambda i: (0, 0)),
            pl.BlockSpec((1, 1), lambda i: (0, 0)),
            pl.BlockSpec((E,), lambda i: (0,)),
        ],
        out_shape=[
            jax.ShapeDtypeStruct((T, D), jnp.float32),
            jax.ShapeDtypeStruct((T,), jnp.int32),
            jax.ShapeDtypeStruct((T,), jnp.int32),
            jax.ShapeDtypeStruct((T, 1), jnp.float32),
            jax.ShapeDtypeStruct((T, 1), jnp.float32),
            jax.ShapeDtypeStruct((1, E), jnp.float32),
            jax.ShapeDtypeStruct((1, E), jnp.float32),
            jax.ShapeDtypeStruct((1, 1), jnp.float32),
            jax.ShapeDtypeStruct((E,), jnp.int32),
        ],
    )(vis, query.reshape(B, 1, QD), ln_g.reshape(1, IN_DIM),
      ln_b.reshape(1, IN_DIM), w_in, b_in.reshape(1, D), w_out,
      b_out.reshape(1, D), w_rx, w_rq, b_r.reshape(1, E))


_SC_MESH = plsc.VectorSubcoreMesh(core_axis_name="c", subcore_axis_name="s",
                                  num_cores=2, num_subcores=16)


def _scatter_body(x3_hbm, s1_hbm, s2_hbm, xg_hbm, idx1_v, idx2_v, rows_v,
                  sem):
    wid = lax.axis_index("s") * 2 + lax.axis_index("c")
    base = wid * TPW
    pltpu.sync_copy(s1_hbm.at[pl.ds(base, TPW)], idx1_v)
    pltpu.sync_copy(s2_hbm.at[pl.ds(base, TPW)], idx2_v)
    pltpu.sync_copy(x3_hbm.at[pl.ds(base, TPW)], rows_v)
    c1 = pltpu.async_copy(rows_v, xg_hbm.at[idx1_v], sem)
    c2 = pltpu.async_copy(rows_v, xg_hbm.at[idx2_v], sem)
    c1.wait()
    c2.wait()


def _sc_scatter(x2, s1, s2):
    return pl.kernel(
        _scatter_body,
        out_type=jax.ShapeDtypeStruct((SLOTS, D), jnp.float32),
        mesh=_SC_MESH,
        scratch_types=[
            pltpu.VMEM((TPW,), jnp.int32),
            pltpu.VMEM((TPW,), jnp.int32),
            pltpu.VMEM((TPW, D), jnp.float32),
            pltpu.SemaphoreType.DMA,
        ],
    )(x2, s1, s2)


def _warm_body(br_hbm, out_hbm, buf_v):
    wid = lax.axis_index("s") * 2 + lax.axis_index("c")
    pltpu.sync_copy(br_hbm, buf_v)
    pltpu.sync_copy(buf_v, out_hbm.at[pl.ds(wid * E, E)])


def _sc_warm(br):
    return pl.kernel(
        _warm_body,
        out_type=jax.ShapeDtypeStruct((NTILES * E,), jnp.float32),
        mesh=_SC_MESH,
        scratch_types=[pltpu.VMEM((E,), jnp.float32)],
    )(br)


def _ffn_body(nb_ref, xg_ref, w1_ref, b1_ref, w2_ref, b2_ref, yg_ref):
    e = pl.program_id(0)
    j = pl.program_id(1)

    @pl.when(j < nb_ref[e])
    def _():
        x = xg_ref[...].astype(jnp.bfloat16)
        h = _gelu(jnp.dot(x, w1_ref[...].reshape(D, DFF).astype(jnp.bfloat16),
                          preferred_element_type=jnp.float32)
                  + b1_ref[...].reshape(1, DFF))
        y = jnp.dot(h.astype(jnp.bfloat16),
                    w2_ref[...].reshape(DFF, D).astype(jnp.bfloat16),
                    preferred_element_type=jnp.float32)
        yg_ref[...] = y + b2_ref[...].reshape(1, D)


def _ffn(nb, xg, W1, b1, W2, b2):
    def xg_map(e, j, nb_ref):
        j_eff = jnp.minimum(j, jnp.maximum(nb_ref[e] - 1, 0))
        return (e * NBLK + j_eff, 0)

    grid_spec = pltpu.PrefetchScalarGridSpec(
        num_scalar_prefetch=1,
        grid=(E, NBLK),
        in_specs=[
            pl.BlockSpec((128, D), xg_map),
            pl.BlockSpec((1, D, DFF), lambda e, j, nb_ref: (e, 0, 0)),
            pl.BlockSpec((1, 1, DFF), lambda e, j, nb_ref: (e, 0, 0)),
            pl.BlockSpec((1, DFF, D), lambda e, j, nb_ref: (e, 0, 0)),
            pl.BlockSpec((1, 1, D), lambda e, j, nb_ref: (e, 0, 0)),
        ],
        out_specs=pl.BlockSpec(
            (128, D),
            lambda e, j, nb_ref: (
                e * NBLK + jnp.minimum(j, jnp.maximum(nb_ref[e] - 1, 0)), 0)),
    )
    return pl.pallas_call(
        _ffn_body,
        grid_spec=grid_spec,
        out_shape=jax.ShapeDtypeStruct((SLOTS, D), jnp.float32),
    )(nb, xg, W1, b1.reshape(E, 1, DFF), W2, b2.reshape(E, 1, D))


def _gather_body(yg_hbm, s1_hbm, s2_hbm, y1_hbm, y2_hbm, idx1_v, idx2_v,
                 buf1_v, buf2_v, sem):
    wid = lax.axis_index("s") * 2 + lax.axis_index("c")
    base = wid * TPW
    pltpu.sync_copy(s1_hbm.at[pl.ds(base, TPW)], idx1_v)
    pltpu.sync_copy(s2_hbm.at[pl.ds(base, TPW)], idx2_v)
    c1 = pltpu.async_copy(yg_hbm.at[idx1_v], buf1_v, sem)
    c2 = pltpu.async_copy(yg_hbm.at[idx2_v], buf2_v, sem)
    c1.wait()
    c2.wait()
    pltpu.sync_copy(buf1_v, y1_hbm.at[pl.ds(base, TPW)])
    pltpu.sync_copy(buf2_v, y2_hbm.at[pl.ds(base, TPW)])


def _sc_gather(yg, s1, s2):
    return pl.kernel(
        _gather_body,
        out_type=[
            jax.ShapeDtypeStruct((T, D), jnp.float32),
            jax.ShapeDtypeStruct((T, D), jnp.float32),
        ],
        mesh=_SC_MESH,
        scratch_types=[
            pltpu.VMEM((TPW,), jnp.int32),
            pltpu.VMEM((TPW,), jnp.int32),
            pltpu.VMEM((TPW, D), jnp.float32),
            pltpu.VMEM((TPW, D), jnp.float32),
            pltpu.SemaphoreType.DMA,
        ],
    )(yg, s1, s2)


def _combine_body(y1_ref, y2_ref, g1_ref, g2_ref, msum_ref, psum_ref,
                  zsum_ref, out_ref, lb_ref, z_ref):
    out_ref[...] = g1_ref[...] * y1_ref[...] + g2_ref[...] * y2_ref[...]
    f = msum_ref[...] * (1.0 / T)
    pm = psum_ref[...] * (1.0 / T)
    lb_ref[...] = (E * jnp.sum(f * pm)).reshape(1, 1)
    z_ref[...] = zsum_ref[...] * (1.0 / T)


def _combine(y1, y2, g1, g2, msum, psum, zsum):
    return pl.pallas_call(
        _combine_body,
        out_shape=[
            jax.ShapeDtypeStruct((T, D), jnp.float32),
            jax.ShapeDtypeStruct((1, 1), jnp.float32),
            jax.ShapeDtypeStruct((1, 1), jnp.float32),
        ],
    )(y1, y2, g1, g2, msum, psum, zsum)


def kernel(vis_emb, query_emb, ln_g, ln_b, w_in, b_in, w_out, b_out,
           w_r, b_r, W1, b1, W2, b2):
    vis = vis_emb.reshape(T, IN_DIM)
    warm = _sc_warm(b_r)
    (xbf, s1f, s2f, g1, g2, msum, psum, zsum, nb) = _mlp1_router(
        vis, query_emb, ln_g, ln_b, w_in, b_in, w_out, b_out,
        w_r[:D], w_r[D:], b_r)
    wz = (warm[0] * 0.0).astype(jnp.int32)
    s1f = s1f + wz
    xg = _sc_scatter(xbf, s1f, s2f)
    yg = _ffn(nb, xg, W1, b1, W2, b2)
    y1, y2 = _sc_gather(yg, s1f, s2f)
    out, lb, z = _combine(y1, y2, g1, g2, msum, psum, zsum)
    return (out.reshape(B, N, D), lb.reshape(()), z.reshape(()))


# speedup vs baseline: 1.9938x; 1.0017x over previous
import jax
import jax.numpy as jnp
from jax import lax
from jax.experimental import pallas as pl
from jax.experimental.pallas import tpu as pltpu
from jax.experimental.pallas import tpu_sc as plsc

B, N = 4, 256
T = B * N
IN_DIM, D, QD = 4096, 1024, 1024
E, K, DFF = 8, 2, 2048
CAP = T
SLOTS = E * CAP
NBLK = CAP // 128
NTILES = 32
TPW = T // NTILES


def _bdot(a, b):
    return jnp.dot(a.astype(jnp.bfloat16), b.astype(jnp.bfloat16),
                   preferred_element_type=jnp.float32)


def _gelu(x):
    return 0.5 * x * (1.0 + jax.lax.erf(x * 0.7071067811865476))


def _mlp1_body(vis_ref, q_ref, lng_ref, lnb_ref, win_ref, bin_ref,
               wout_ref, bout_ref, wrx_ref, wrq_ref, br_ref,
               xbf_ref, s1_ref, s2_ref, g1_ref, g2_ref,
               mask_sum_ref, p_sum_ref, z_sum_ref, nb_ref):
    i = pl.program_id(0)
    v = vis_ref[...]
    mu = jnp.mean(v, axis=-1, keepdims=True)
    var = jnp.mean((v - mu) ** 2, axis=-1, keepdims=True)
    ln = (v - mu) / jnp.sqrt(var + 1e-5) * lng_ref[...] + lnb_ref[...]
    h = _gelu(_bdot(ln, win_ref[...]) + bin_ref[...])
    x = _bdot(h, wout_ref[...]) + bout_ref[...]
    xbf_ref[...] = x
    qlog = _bdot(q_ref[...].reshape(1, QD), wrq_ref[...])
    logits = _bdot(x, wrx_ref[...]) + qlog + br_ref[...]

    m = jnp.max(logits, axis=-1, keepdims=True)
    unnorm = jnp.exp(logits - m)
    denom = jnp.sum(unnorm, axis=-1, keepdims=True)
    probs = unnorm / denom
    lse = jnp.log(denom) + m

    eio = jax.lax.broadcasted_iota(jnp.int32, (N, E), 1)
    m1 = jnp.max(probs, axis=-1, keepdims=True)
    i1 = jnp.min(jnp.where(probs == m1, eio, E), axis=-1, keepdims=True)
    masked = jnp.where(eio == i1, -1.0, probs)
    m2 = jnp.max(masked, axis=-1, keepdims=True)
    i2 = jnp.min(jnp.where(masked == m2, eio, E), axis=-1, keepdims=True)
    tot = m1 + m2
    g1_ref[...] = m1 / tot
    g2_ref[...] = m2 / tot
    sel1 = eio == i1
    sel2 = eio == i2
    mask = sel1.astype(jnp.float32) + sel2.astype(jnp.float32)

    @pl.when(i == 0)
    def _():
        mask_sum_ref[...] = jnp.zeros_like(mask_sum_ref)
        p_sum_ref[...] = jnp.zeros_like(p_sum_ref)
        z_sum_ref[...] = jnp.zeros_like(z_sum_ref)

    rio = jax.lax.broadcasted_iota(jnp.int32, (N, N), 0)
    cio = jax.lax.broadcasted_iota(jnp.int32, (N, N), 1)
    lt = (cio < rio).astype(jnp.bfloat16)
    cum = jnp.dot(lt, mask.astype(jnp.bfloat16),
                  preferred_element_type=jnp.float32)
    rmat = mask_sum_ref[...] + cum
    r1 = jnp.sum(jnp.where(sel1, rmat, 0.0), axis=-1, keepdims=True)
    r2 = jnp.sum(jnp.where(sel2, rmat, 0.0), axis=-1, keepdims=True)
    s1_ref[...] = (i1 * CAP + r1.astype(jnp.int32)).reshape(N)
    s2_ref[...] = (i2 * CAP + r2.astype(jnp.int32)).reshape(N)

    mask_sum_ref[...] += jnp.sum(mask, axis=0, keepdims=True)
    p_sum_ref[...] += jnp.sum(probs, axis=0, keepdims=True)
    z_sum_ref[...] += jnp.sum(lse * lse).reshape(1, 1)
    cnt = mask_sum_ref[...].astype(jnp.int32)
    nb_ref[...] = jax.lax.shift_right_logical(cnt + 127, 7).reshape(E)


def _mlp1_router(vis, query, ln_g, ln_b, w_in, b_in, w_out, b_out,
                 w_rx, w_rq, b_r):
    return pl.pallas_call(
        _mlp1_body,
        grid=(B,),
        in_specs=[
            pl.BlockSpec((N, IN_DIM), lambda i: (i, 0)),
            pl.BlockSpec((1, 1, QD), lambda i: (i, 0, 0)),
            pl.BlockSpec((1, IN_DIM), lambda i: (0, 0)),
            pl.BlockSpec((1, IN_DIM), lambda i: (0, 0)),
            pl.BlockSpec((IN_DIM, D), lambda i: (0, 0)),
            pl.BlockSpec((1, D), lambda i: (0, 0)),
            pl.BlockSpec((D, D), lambda i: (0, 0)),
            pl.BlockSpec((1, D), lambda i: (0, 0)),
            pl.BlockSpec((D, E), lambda i: (0, 0)),
            pl.BlockSpec((QD, E), lambda i: (0, 0)),
            pl.BlockSpec((1, E), lambda i: (0, 0)),
        ],
        out_specs=[
            pl.BlockSpec((N, D), lambda i: (i, 0)),
            pl.BlockSpec((N,), lambda i: (i,)),
            pl.BlockSpec((N,), lambda i: (i,)),
            pl.BlockSpec((N, 1), lambda i: (i, 0)),
            pl.BlockSpec((N, 1), lambda i: (i, 0)),
            pl.BlockSpec((1, E), lambda i: (0, 0)),
            pl.BlockSpec((1, E), lambda i: (0, 0)),
            pl.BlockSpec((1, 1), lambda i: (0, 0)),
            pl.BlockSpec((E,), lambda i: (0,)),
        ],
        out_shape=[
            jax.ShapeDtypeStruct((T, D), jnp.float32),
            jax.ShapeDtypeStruct((T,), jnp.int32),
            jax.ShapeDtypeStruct((T,), jnp.int32),
            jax.ShapeDtypeStruct((T, 1), jnp.float32),
            jax.ShapeDtypeStruct((T, 1), jnp.float32),
            jax.ShapeDtypeStruct((1, E), jnp.float32),
            jax.ShapeDtypeStruct((1, E), jnp.float32),
            jax.ShapeDtypeStruct((1, 1), jnp.float32),
            jax.ShapeDtypeStruct((E,), jnp.int32),
        ],
    )(vis, query.reshape(B, 1, QD), ln_g.reshape(1, IN_DIM),
      ln_b.reshape(1, IN_DIM), w_in, b_in.reshape(1, D), w_out,
      b_out.reshape(1, D), w_rx, w_rq, b_r.reshape(1, E))


_SC_MESH = plsc.VectorSubcoreMesh(core_axis_name="c", subcore_axis_name="s",
                                  num_cores=2, num_subcores=16)


def _scatter_body(x3_hbm, s1_hbm, s2_hbm, xg_hbm, idx1_v, idx2_v, rows_v,
                  sem):
    wid = lax.axis_index("s") * 2 + lax.axis_index("c")
    base = wid * TPW
    pltpu.sync_copy(s1_hbm.at[pl.ds(base, TPW)], idx1_v)
    pltpu.sync_copy(s2_hbm.at[pl.ds(base, TPW)], idx2_v)
    pltpu.sync_copy(x3_hbm.at[pl.ds(base, TPW)], rows_v)
    c1 = pltpu.async_copy(rows_v, xg_hbm.at[idx1_v], sem)
    c2 = pltpu.async_copy(rows_v, xg_hbm.at[idx2_v], sem)
    c1.wait()
    c2.wait()


def _sc_scatter(x2, s1, s2):
    return pl.kernel(
        _scatter_body,
        out_type=jax.ShapeDtypeStruct((SLOTS, D), jnp.float32),
        mesh=_SC_MESH,
        scratch_types=[
            pltpu.VMEM((TPW,), jnp.int32),
            pltpu.VMEM((TPW,), jnp.int32),
            pltpu.VMEM((TPW, D), jnp.float32),
            pltpu.SemaphoreType.DMA,
        ],
    )(x2, s1, s2)


def _warm_body(br_hbm, out_hbm, buf_v):
    wid = lax.axis_index("s") * 2 + lax.axis_index("c")
    pltpu.sync_copy(br_hbm, buf_v)
    pltpu.sync_copy(buf_v, out_hbm.at[pl.ds(wid * E, E)])


def _sc_warm(br):
    return pl.kernel(
        _warm_body,
        out_type=jax.ShapeDtypeStruct((NTILES * E,), jnp.float32),
        mesh=_SC_MESH,
        scratch_types=[pltpu.VMEM((E,), jnp.float32)],
    )(br)


def _ffn_body(nb_ref, xg_ref, w1_ref, b1_ref, w2_ref, b2_ref, yg_ref):
    e = pl.program_id(0)
    j = pl.program_id(1)

    @pl.when(j < nb_ref[e])
    def _():
        x = xg_ref[...].astype(jnp.bfloat16)
        h = _gelu(jnp.dot(x, w1_ref[...].reshape(D, DFF).astype(jnp.bfloat16),
                          preferred_element_type=jnp.float32)
                  + b1_ref[...].reshape(1, DFF))
        y = jnp.dot(h.astype(jnp.bfloat16),
                    w2_ref[...].reshape(DFF, D).astype(jnp.bfloat16),
                    preferred_element_type=jnp.float32)
        yg_ref[...] = y + b2_ref[...].reshape(1, D)


def _ffn(nb, xg, W1, b1, W2, b2):
    def xg_map(e, j, nb_ref):
        j_eff = jnp.minimum(j, jnp.maximum(nb_ref[e] - 1, 0))
        return (e * NBLK + j_eff, 0)

    grid_spec = pltpu.PrefetchScalarGridSpec(
        num_scalar_prefetch=1,
        grid=(E, NBLK),
        in_specs=[
            pl.BlockSpec((128, D), xg_map),
            pl.BlockSpec((1, D, DFF), lambda e, j, nb_ref: (e, 0, 0)),
            pl.BlockSpec((1, 1, DFF), lambda e, j, nb_ref: (e, 0, 0)),
            pl.BlockSpec((1, DFF, D), lambda e, j, nb_ref: (e, 0, 0)),
            pl.BlockSpec((1, 1, D), lambda e, j, nb_ref: (e, 0, 0)),
        ],
        out_specs=pl.BlockSpec(
            (128, D),
            lambda e, j, nb_ref: (
                e * NBLK + jnp.minimum(j, jnp.maximum(nb_ref[e] - 1, 0)), 0)),
    )
    return pl.pallas_call(
        _ffn_body,
        grid_spec=grid_spec,
        out_shape=jax.ShapeDtypeStruct((SLOTS, D), jnp.float32),
    )(nb, xg, W1, b1.reshape(E, 1, DFF), W2, b2.reshape(E, 1, D))


def _gather_body(yg_hbm, s1_hbm, s2_hbm, y1_hbm, y2_hbm, idx1_v, idx2_v,
                 buf1_v, buf2_v, sem):
    wid = lax.axis_index("s") * 2 + lax.axis_index("c")
    base = wid * TPW
    pltpu.sync_copy(s1_hbm.at[pl.ds(base, TPW)], idx1_v)
    pltpu.sync_copy(s2_hbm.at[pl.ds(base, TPW)], idx2_v)
    c1 = pltpu.async_copy(yg_hbm.at[idx1_v], buf1_v, sem)
    c2 = pltpu.async_copy(yg_hbm.at[idx2_v], buf2_v, sem)
    c1.wait()
    c2.wait()
    pltpu.sync_copy(buf1_v, y1_hbm.at[pl.ds(base, TPW)])
    pltpu.sync_copy(buf2_v, y2_hbm.at[pl.ds(base, TPW)])


def _sc_gather(yg, s1, s2):
    return pl.kernel(
        _gather_body,
        out_type=[
            jax.ShapeDtypeStruct((T, D), jnp.float32),
            jax.ShapeDtypeStruct((T, D), jnp.float32),
        ],
        mesh=_SC_MESH,
        scratch_types=[
            pltpu.VMEM((TPW,), jnp.int32),
            pltpu.VMEM((TPW,), jnp.int32),
            pltpu.VMEM((TPW, D), jnp.float32),
            pltpu.VMEM((TPW, D), jnp.float32),
            pltpu.SemaphoreType.DMA,
        ],
    )(yg, s1, s2)


def _combine_body(y1_ref, y2_ref, g1_ref, g2_ref, msum_ref, psum_ref,
                  zsum_ref, out_ref, lb_ref, z_ref):
    out_ref[...] = g1_ref[...] * y1_ref[...] + g2_ref[...] * y2_ref[...]
    f = msum_ref[...] * (1.0 / T)
    pm = psum_ref[...] * (1.0 / T)
    lb_ref[...] = (E * jnp.sum(f * pm)).reshape(1, 1)
    z_ref[...] = zsum_ref[...] * (1.0 / T)


def _combine(y1, y2, g1, g2, msum, psum, zsum):
    return pl.pallas_call(
        _combine_body,
        out_shape=[
            jax.ShapeDtypeStruct((T, D), jnp.float32),
            jax.ShapeDtypeStruct((1, 1), jnp.float32),
            jax.ShapeDtypeStruct((1, 1), jnp.float32),
        ],
    )(y1, y2, g1, g2, msum, psum, zsum)


def kernel(vis_emb, query_emb, ln_g, ln_b, w_in, b_in, w_out, b_out,
           w_r, b_r, W1, b1, W2, b2):
    vis = vis_emb.reshape(T, IN_DIM)
    warm = _sc_warm(b_r)
    (xbf, s1f, s2f, g1, g2, msum, psum, zsum, nb) = _mlp1_router(
        vis, query_emb, ln_g, ln_b, w_in, b_in, w_out, b_out,
        w_r[:D], w_r[D:], b_r)
    wz = (warm[0] * 0.0).astype(jnp.int32)
    s1f = s1f + wz
    xg = _sc_scatter(xbf, s1f, s2f)
    yg = _ffn(nb, xg, W1, b1, W2, b2)
    y1, y2 = _sc_gather(yg, s1f, s2f)
    out, lb, z = _combine(y1, y2, g1, g2, msum, psum, zsum)
    return (out.reshape(B, N, D), lb.reshape(()), z.reshape(()))
